# baseline (speedup 1.0000x reference)
import numpy as np
import jax
import jax.numpy as jnp
from jax import lax
from jax.experimental import pallas as pl
from jax.experimental.pallas import tpu as pltpu
from jax.experimental.pallas import tpu_sc as plsc

N = 10000
E = 160000
D = 256
G = 64
EPS = 1e-5

NC = 2
NS = 16
HALF = D // NC
W = 128
WPS = 80
WPD = 40
WCH = 16
NWIN_P = NS * WPS
E_PAD = NWIN_P * W
NP = 10240
STRIPE = NP // NS

BLK = 400
NBLK = N // BLK

_mesh = plsc.VectorSubcoreMesh(
    core_axis_name="c", subcore_axis_name="s", num_cores=NC, num_subcores=NS
)

_CHUNKS = [(o, min(128, STRIPE - o)) for o in range(0, STRIPE, 128)]


def _fill_f32(ref, rows, cols, val):
    @pl.loop(0, rows)
    def _(i):
        for j in range(cols // 16):
            ref[i, pl.ds(j * 16, 16)] = jnp.full((16,), val, jnp.float32)


def _wait_dma(hbm_ref, vmem_ref, sem):
    pltpu.make_async_copy(hbm_ref.at[pl.ds(0, W)], vmem_ref, sem).wait()


def _segsum_body(t_hbm, src_hbm, dst_hbm, out_hbm, srcb, dstb,
                 r0, r1, acc, g0, g1, s0, s1):
    c = lax.axis_index("c")
    s = lax.axis_index("s")
    base = s * STRIPE
    rows = [r0, r1]
    gsem = [g0, g1]
    ssem = [s0, s1]

    _fill_f32(r0, W, HALF, 0.0)
    for o, sz in _CHUNKS:
        pltpu.sync_copy(r0.at[pl.ds(0, sz)], acc.at[pl.ds(base + o, sz)])
    plsc.subcore_barrier()

    row_off = c * NP

    def start_gather(i, b):
        pltpu.async_copy(t_hbm.at[srcb.at[i]], rows[b], gsem[b])

    def start_scatter(j, d):
        pltpu.async_copy(rows[d], acc.at[dstb.at[j]], ssem[d], add=True)

    @pl.loop(0, WPS // WCH)
    def _(p):
        base_w = s * WPS + p * WCH
        pltpu.sync_copy(src_hbm.at[pl.ds(base_w, WCH)], srcb)
        pltpu.sync_copy(dst_hbm.at[pl.ds(base_w, WCH)], dstb)

        @pl.loop(0, WCH)
        def _(i):
            for j in range(W // 16):
                sl = pl.ds(j * 16, 16)
                srcb[i, sl] = srcb[i, sl] + row_off

        @pl.loop(0, WCH // 2)
        def _(k):
            for b in range(2):
                @pl.when(k > 0)
                def _():
                    _wait_dma(t_hbm, rows[b], ssem[b])
                start_gather(k * 2 + b, b)
                d = (b + 1) % 2
                if b == 0:
                    @pl.when(k > 0)
                    def _():
                        _wait_dma(t_hbm, rows[d], gsem[d])
                        start_scatter(k * 2 - 1, d)
                else:
                    _wait_dma(t_hbm, rows[d], gsem[d])
                    start_scatter(k * 2, d)

        _wait_dma(t_hbm, rows[1], gsem[1])
        start_scatter(WCH - 1, 1)
        for b in range(2):
            _wait_dma(t_hbm, rows[b], ssem[b])

    plsc.subcore_barrier()
    for o, sz in _CHUNKS:
        pltpu.sync_copy(
            acc.at[pl.ds(base + o, sz)],
            out_hbm.at[pl.ds(c * NP + base + o, sz)],
        )


_segsum = pl.kernel(
    _segsum_body,
    out_type=jax.ShapeDtypeStruct((NC * NP, HALF), jnp.float32),
    mesh=_mesh,
    scratch_types=[
        pltpu.VMEM((WCH, W), jnp.int32),
        pltpu.VMEM((WCH, W), jnp.int32),
        pltpu.VMEM((W, HALF), jnp.float32),
        pltpu.VMEM((W, HALF), jnp.float32),
        pltpu.VMEM_SHARED((NP, HALF), jnp.float32),
        pltpu.SemaphoreType.DMA,
        pltpu.SemaphoreType.DMA,
        pltpu.SemaphoreType.DMA,
        pltpu.SemaphoreType.DMA,
    ],
)


def _deg_body(dst_hbm, out_hbm, dstb, ones, accd, s0, s1, s2, s3):
    c = lax.axis_index("c")
    s = lax.axis_index("s")
    base = s * STRIPE
    ssem = [s0, s1, s2, s3]

    _fill_f32(ones, W, HALF, 0.0)
    for o, sz in _CHUNKS:
        pltpu.sync_copy(ones.at[pl.ds(0, sz)], accd.at[pl.ds(base + o, sz)])
    _fill_f32(ones, W, HALF, 1.0)
    pltpu.sync_copy(dst_hbm.at[pl.ds((c * NS + s) * WPD, WPD)], dstb)
    plsc.subcore_barrier()

    @pl.loop(0, WPD // 4)
    def _(k):
        for b in range(4):
            @pl.when(k > 0)
            def _():
                _wait_dma(out_hbm, ones, ssem[b])
            pltpu.async_copy(ones, accd.at[dstb.at[k * 4 + b]], ssem[b],
                             add=True)

    for b in range(4):
        _wait_dma(out_hbm, ones, ssem[b])

    plsc.subcore_barrier()
    for o, sz in _CHUNKS:
        pltpu.sync_copy(
            accd.at[pl.ds(base + o, sz)],
            out_hbm.at[pl.ds(c * NP + base + o, sz)],
        )


_deg = pl.kernel(
    _deg_body,
    out_type=jax.ShapeDtypeStruct((NC * NP, HALF), jnp.float32),
    mesh=_mesh,
    scratch_types=[
        pltpu.VMEM((WPD, W), jnp.int32),
        pltpu.VMEM((W, HALF), jnp.float32),
        pltpu.VMEM_SHARED((NP, HALF), jnp.float32),
        pltpu.SemaphoreType.DMA,
        pltpu.SemaphoreType.DMA,
        pltpu.SemaphoreType.DMA,
        pltpu.SemaphoreType.DMA,
    ],
)


def _mm0_body(x_ref, wl_ref, wr_ref, t_ref, u_ref):
    xb = x_ref[...]
    t = jnp.dot(xb, wl_ref[...], preferred_element_type=jnp.float32)
    t_ref[0] = t[:, :HALF]
    t_ref[1] = t[:, HALF:]
    u_ref[...] = jnp.dot(xb, wr_ref[...], preferred_element_type=jnp.float32)


def _mm0(x, wl, wr):
    return pl.pallas_call(
        _mm0_body,
        grid=(NBLK,),
        in_specs=[
            pl.BlockSpec((BLK, D), lambda i: (i, 0)),
            pl.BlockSpec((D, D), lambda i: (0, 0)),
            pl.BlockSpec((D, D), lambda i: (0, 0)),
        ],
        out_specs=[
            pl.BlockSpec((NC, BLK, HALF), lambda i: (0, i, 0)),
            pl.BlockSpec((BLK, D), lambda i: (i, 0)),
        ],
        out_shape=[
            jax.ShapeDtypeStruct((NC, NP, HALF), jnp.float32),
            jax.ShapeDtypeStruct((N, D), jnp.float32),
        ],
    )(x, wl, wr)


def _act(s_ref, u_ref, deg_ref, b_ref, g_ref, be_ref):
    sfull = jnp.concatenate([s_ref[0], s_ref[1]], axis=1)
    deg = deg_ref[0, :, 0:1] + deg_ref[1, :, 0:1]
    agg = sfull / jnp.maximum(deg, 1.0)
    h = agg + u_ref[...] + b_ref[...]
    h = g_ref[...] * (h * (1.0 / np.sqrt(1.0 + EPS))) + be_ref[...]
    return jnp.maximum(h, 0.0)


def _mid_body(s_ref, u_ref, deg_ref, b_ref, g_ref, be_ref, wl_ref, wr_ref,
              t_ref, u2_ref):
    h = _act(s_ref, u_ref, deg_ref, b_ref, g_ref, be_ref)
    t = jnp.dot(h, wl_ref[...], preferred_element_type=jnp.float32)
    t_ref[0] = t[:, :HALF]
    t_ref[1] = t[:, HALF:]
    u2_ref[...] = jnp.dot(h, wr_ref[...], preferred_element_type=jnp.float32)


def _mid(s_, u, deg2, b, g, be, wl, wr):
    return pl.pallas_call(
        _mid_body,
        grid=(NBLK,),
        in_specs=[
            pl.BlockSpec((NC, BLK, HALF), lambda i: (0, i, 0)),
            pl.BlockSpec((BLK, D), lambda i: (i, 0)),
            pl.BlockSpec((NC, BLK, HALF), lambda i: (0, i, 0)),
            pl.BlockSpec((1, D), lambda i: (0, 0)),
            pl.BlockSpec((1, D), lambda i: (0, 0)),
            pl.BlockSpec((1, D), lambda i: (0, 0)),
            pl.BlockSpec((D, D), lambda i: (0, 0)),
            pl.BlockSpec((D, D), lambda i: (0, 0)),
        ],
        out_specs=[
            pl.BlockSpec((NC, BLK, HALF), lambda i: (0, i, 0)),
            pl.BlockSpec((BLK, D), lambda i: (i, 0)),
        ],
        out_shape=[
            jax.ShapeDtypeStruct((NC, NP, HALF), jnp.float32),
            jax.ShapeDtypeStruct((N, D), jnp.float32),
        ],
    )(s_, u, deg2, b, g, be, wl, wr)


def _final_body(s_ref, u_ref, deg_ref, b_ref, g_ref, be_ref, batch_ref,
                out_ref, acc_ref, cnt_ref):
    i = pl.program_id(0)

    @pl.when(i == 0)
    def _():
        acc_ref[...] = jnp.zeros((G, D), jnp.float32)
        cnt_ref[...] = jnp.zeros((G, 1), jnp.float32)

    h = _act(s_ref, u_ref, deg_ref, b_ref, g_ref, be_ref)
    bt = batch_ref[0, 0, :]
    onehot = (bt[None, :] == lax.broadcasted_iota(jnp.int32, (G, BLK), 0))
    onehot = onehot.astype(jnp.float32)
    acc_ref[...] += jnp.dot(onehot, h, preferred_element_type=jnp.float32)
    cnt_ref[...] += jnp.sum(onehot, axis=1, keepdims=True)

    @pl.when(i == NBLK - 1)
    def _():
        out_ref[...] = acc_ref[...] / jnp.maximum(cnt_ref[...], 1.0)


def _final(s_, u, deg2, b, g, be, batch3):
    return pl.pallas_call(
        _final_body,
        grid=(NBLK,),
        in_specs=[
            pl.BlockSpec((NC, BLK, HALF), lambda i: (0, i, 0)),
            pl.BlockSpec((BLK, D), lambda i: (i, 0)),
            pl.BlockSpec((NC, BLK, HALF), lambda i: (0, i, 0)),
            pl.BlockSpec((1, D), lambda i: (0, 0)),
            pl.BlockSpec((1, D), lambda i: (0, 0)),
            pl.BlockSpec((1, D), lambda i: (0, 0)),
            pl.BlockSpec((1, 1, BLK), lambda i: (i, 0, 0)),
        ],
        out_specs=pl.BlockSpec((G, D), lambda i: (0, 0)),
        out_shape=jax.ShapeDtypeStruct((G, D), jnp.float32),
        scratch_shapes=[
            pltpu.VMEM((G, D), jnp.float32),
            pltpu.VMEM((G, 1), jnp.float32),
        ],
    )(s_, u, deg2, b, g, be, batch3)


def kernel(x, edge_index, batch, Wl0, Wr0, b0, gamma0, beta0,
           Wl1, Wr1, b1, gamma1, beta1, Wl2, Wr2, b2, gamma2, beta2):
    src = edge_index[0]
    dst = edge_index[1]
    pad = E_PAD - E
    src2d = jnp.concatenate([src, jnp.zeros((pad,), jnp.int32)]).reshape(NWIN_P, W)
    dst2d = jnp.concatenate([dst, jnp.full((pad,), N, jnp.int32)]).reshape(NWIN_P, W)
    batch3 = batch.reshape(NBLK, 1, BLK)
    params = [
        (b0.reshape(1, D), gamma0.reshape(1, D), beta0.reshape(1, D), Wl1, Wr1),
        (b1.reshape(1, D), gamma1.reshape(1, D), beta1.reshape(1, D), Wl2, Wr2),
        (b2.reshape(1, D), gamma2.reshape(1, D), beta2.reshape(1, D), None, None),
    ]

    deg2 = _deg(dst2d).reshape(NC, NP, HALF)
    t, u = _mm0(x, Wl0, Wr0)
    for li in range(3):
        s_ = _segsum(t.reshape(NC * NP, HALF), src2d, dst2d).reshape(NC, NP, HALF)
        b, g, be, wl, wr = params[li]
        if li < 2:
            t, u = _mid(s_, u, deg2, b, g, be, wl, wr)
        else:
            return _final(s_, u, deg2, b, g, be, batch3)
